# baseline (speedup 1.0000x reference)
import functools

import jax
import jax.numpy as jnp
from jax import lax
from jax.experimental import pallas as pl
from jax.experimental.pallas import tpu as pltpu
from jax.experimental.pallas import tpu_sc as plsc

B, L = 4096, 50
D0, D1, D2 = 64, 32, 32
DOUT = D0 + D1 + D2
N = B * L

_info = plsc.get_sparse_core_info()
NC, NS, LANES = _info.num_cores, _info.num_subcores, _info.num_lanes
NW = NC * NS
PER_W = N // NW
CH = 400
N_CHUNKS = PER_W // CH
NBUF = 2

_GATHER_DNUMS = lax.GatherDimensionNumbers(
    offset_dims=(), collapsed_slice_dims=(0,), start_index_map=(0,))


def _sc_body(ids0_hbm, ids1_hbm, ids2_hbm, mask_hbm, w0_hbm, w1_hbm, w2_hbm,
             out_hbm, idx0_v, idx1_v, idx2_v, mask_v, e0_v, e1_v, e2_v,
             gsem, wsem):
    wid = lax.axis_index("s") * NC + lax.axis_index("c")
    tok_w = wid * PER_W

    pltpu.sync_copy(ids0_hbm.at[pl.ds(tok_w, PER_W)], idx0_v)
    pltpu.sync_copy(ids1_hbm.at[pl.ds(tok_w, PER_W)], idx1_v)
    pltpu.sync_copy(ids2_hbm.at[pl.ds(tok_w, PER_W)], idx2_v)
    pltpu.sync_copy(mask_hbm.at[pl.ds(tok_w, PER_W)], mask_v)

    def fire_gathers(ci, s):
        rows = pl.ds(ci * CH, CH)
        pltpu.async_copy(w0_hbm.at[idx0_v.at[rows]], e0_v.at[s], gsem)
        pltpu.async_copy(w1_hbm.at[idx1_v.at[rows]], e1_v.at[s], gsem)
        pltpu.async_copy(w2_hbm.at[idx2_v.at[rows]], e2_v.at[s], gsem)

    def wait_gathers(s):
        pltpu.make_async_copy(w0_hbm.at[pl.ds(0, CH)], e0_v.at[s],
                              gsem).wait()
        pltpu.make_async_copy(w1_hbm.at[pl.ds(0, CH)], e1_v.at[s],
                              gsem).wait()
        pltpu.make_async_copy(w1_hbm.at[pl.ds(0, CH)], e2_v.at[s],
                              gsem).wait()

    def apply_mask(ci, s):
        def ones_check(g, acc):
            m16 = mask_v[pl.ds(ci * CH + g * LANES, LANES)]
            return acc + jnp.abs(m16 - 1.0)

        dev = lax.fori_loop(0, CH // LANES, ones_check,
                            jnp.zeros((LANES,), dtype=jnp.float32))
        for stride in (1, 2, 4, 8):
            perm = jnp.arange(LANES, dtype=jnp.int32) ^ stride
            dev = dev + lax.gather(
                dev, perm[:, None], _GATHER_DNUMS, slice_sizes=(1,),
                mode=lax.GatherScatterMode.PROMISE_IN_BOUNDS)
        not_ones = dev[0] != 0.0

        @pl.when(not_ones)
        def _():
            def group_body(g, gcarry):
                m16 = mask_v[pl.ds(ci * CH + g * LANES, LANES)]
                for lane in range(LANES):
                    r = g * LANES + lane
                    m = lax.gather(
                        m16, jnp.full((LANES, 1), lane, dtype=jnp.int32),
                        _GATHER_DNUMS, slice_sizes=(1,),
                        mode=lax.GatherScatterMode.PROMISE_IN_BOUNDS)
                    for j in range(D0 // LANES):
                        e0_v[s, r, pl.ds(j * LANES, LANES)] = (
                            e0_v[s, r, pl.ds(j * LANES, LANES)] * m)
                    for j in range(D1 // LANES):
                        e1_v[s, r, pl.ds(j * LANES, LANES)] = (
                            e1_v[s, r, pl.ds(j * LANES, LANES)] * m)
                    for j in range(D2 // LANES):
                        e2_v[s, r, pl.ds(j * LANES, LANES)] = (
                            e2_v[s, r, pl.ds(j * LANES, LANES)] * m)
                return gcarry

            lax.fori_loop(0, CH // LANES, group_body, 0)

    def chunk_writes(ci, s):
        rows = pl.ds(tok_w + ci * CH, CH)
        yield (e0_v.at[s], out_hbm.at[rows, pl.ds(0, D0)])
        yield (e1_v.at[s], out_hbm.at[rows, pl.ds(D0, D1)])
        yield (e2_v.at[s], out_hbm.at[rows, pl.ds(D0 + D1, D2)])

    def fire_writes(ci, s):
        for src, dst in chunk_writes(ci, s):
            pltpu.async_copy(src, dst, wsem)

    def drain_writes(ci, s):
        for src, dst in chunk_writes(ci, s):
            pltpu.make_async_copy(src, dst, wsem).wait()

    fire_gathers(0, 0)

    def pair_body(p, carry):
        for s in range(NBUF):
            ci = p * NBUF + s
            wait_gathers(s)

            @pl.when(ci + 1 < N_CHUNKS)
            def _():
                fire_gathers(ci + 1, s ^ 1)

            @pl.when(ci > 1)
            def _():
                drain_writes(ci - 2, s)

            apply_mask(ci, s)
            fire_writes(ci, s)
        return carry

    lax.fori_loop(0, N_CHUNKS // NBUF, pair_body, 0)
    drain_writes(N_CHUNKS - 2, 0)
    drain_writes(N_CHUNKS - 1, 1)


_sc_call = functools.partial(
    pl.kernel,
    out_type=jax.ShapeDtypeStruct((N, DOUT), jnp.float32),
    mesh=plsc.VectorSubcoreMesh(core_axis_name="c", subcore_axis_name="s"),
    compiler_params=pltpu.CompilerParams(use_tc_tiling_on_sc=False),
    scratch_types=[
        pltpu.VMEM((PER_W,), jnp.int32),
        pltpu.VMEM((PER_W,), jnp.int32),
        pltpu.VMEM((PER_W,), jnp.int32),
        pltpu.VMEM((PER_W,), jnp.float32),
        pltpu.VMEM((NBUF, CH, D0), jnp.float32),
        pltpu.VMEM((NBUF, CH, D1), jnp.float32),
        pltpu.VMEM((NBUF, CH, D2), jnp.float32),
        pltpu.SemaphoreType.DMA,
        pltpu.SemaphoreType.DMA,
    ],
)(_sc_body)


def kernel(positional_ids_0, positional_ids_1, positional_ids_2,
           attention_mask, W0, W1, W2):
    ids0 = positional_ids_0.reshape(N).astype(jnp.int32)
    ids1 = positional_ids_1.reshape(N).astype(jnp.int32)
    ids2 = positional_ids_2.reshape(N).astype(jnp.int32)
    mask = attention_mask.reshape(N).astype(jnp.float32)
    out = _sc_call(ids0, ids1, ids2, mask, W0, W1, W2)
    return out.reshape(B, L, DOUT)


# speedup vs baseline: 1.0720x; 1.0720x over previous
import functools

import jax
import jax.numpy as jnp
from jax import lax
from jax.experimental import pallas as pl
from jax.experimental.pallas import tpu as pltpu
from jax.experimental.pallas import tpu_sc as plsc

B, L = 4096, 50
LP = 56
D0, D1, D2 = 64, 32, 32
DOUT = D0 + D1 + D2
N = B * L
NSLICE = 2
BH = B // NSLICE

_info = plsc.get_sparse_core_info()
NC, NS, LANES = _info.num_cores, _info.num_subcores, _info.num_lanes
NW = NC * NS
BATCH_PER_W = BH // NW
PER_W = BATCH_PER_W * L
CB = 8
CH = CB * L
N_CHUNKS = BATCH_PER_W // CB
NBUF = 2


def _make_sc_body(half):
    def _sc_body(ids0_hbm, ids1_hbm, ids2_hbm, w0_hbm, w1_hbm, w2_hbm,
                 out_hbm, idx0_v, idx1_v, idx2_v, e0_v, e1_v, e2_v,
                 gsem, wsem):
        wid = lax.axis_index("s") * NC + lax.axis_index("c")
        batch_w = wid * BATCH_PER_W
        tok_w = (half * BH + batch_w) * L

        pltpu.sync_copy(ids0_hbm.at[pl.ds(tok_w, PER_W)], idx0_v)
        pltpu.sync_copy(ids1_hbm.at[pl.ds(tok_w, PER_W)], idx1_v)
        pltpu.sync_copy(ids2_hbm.at[pl.ds(tok_w, PER_W)], idx2_v)

        def fire_gathers(ci, s):
            rows = pl.ds(ci * CH, CH)
            pltpu.async_copy(w0_hbm.at[idx0_v.at[rows]], e0_v.at[s], gsem)
            pltpu.async_copy(w1_hbm.at[idx1_v.at[rows]], e1_v.at[s], gsem)
            pltpu.async_copy(w2_hbm.at[idx2_v.at[rows]], e2_v.at[s], gsem)

        def wait_gathers(s):
            pltpu.make_async_copy(w0_hbm.at[pl.ds(0, CH)], e0_v.at[s],
                                  gsem).wait()
            pltpu.make_async_copy(w1_hbm.at[pl.ds(0, CH)], e1_v.at[s],
                                  gsem).wait()
            pltpu.make_async_copy(w1_hbm.at[pl.ds(0, CH)], e2_v.at[s],
                                  gsem).wait()

        def batch_writes(ci, s):
            b0 = batch_w + ci * CB
            for bi in range(CB):
                rows = pl.ds(bi * L, L)
                yield (e0_v.at[s, rows], out_hbm.at[b0 + bi, pl.ds(0, L),
                                                    pl.ds(0, D0)])
                yield (e1_v.at[s, rows], out_hbm.at[b0 + bi, pl.ds(0, L),
                                                    pl.ds(D0, D1)])
                yield (e2_v.at[s, rows], out_hbm.at[b0 + bi, pl.ds(0, L),
                                                    pl.ds(D0 + D1, D2)])

        def fire_writes(ci, s):
            for src, dst in batch_writes(ci, s):
                pltpu.async_copy(src, dst, wsem)

        def drain_writes(ci, s):
            for src, dst in batch_writes(ci, s):
                pltpu.make_async_copy(src, dst, wsem).wait()

        fire_gathers(0, 0)

        def pair_body(p, carry):
            for s in range(NBUF):
                ci = p * NBUF + s
                wait_gathers(s)
                fire_writes(ci, s)

                @pl.when(ci > 0)
                def _():
                    drain_writes(ci - 1, s ^ 1)

                @pl.when(ci + 1 < N_CHUNKS)
                def _():
                    fire_gathers(ci + 1, s ^ 1)
            return carry

        lax.fori_loop(0, N_CHUNKS // NBUF, pair_body, 0)
        drain_writes(N_CHUNKS - 1, (N_CHUNKS - 1) % NBUF)

    return _sc_body


def _make_sc_call(half):
    return functools.partial(
        pl.kernel,
        out_type=jax.ShapeDtypeStruct((BH, LP, DOUT), jnp.float32),
        mesh=plsc.VectorSubcoreMesh(core_axis_name="c", subcore_axis_name="s"),
        compiler_params=pltpu.CompilerParams(use_tc_tiling_on_sc=False),
        scratch_types=[
            pltpu.VMEM((PER_W,), jnp.int32),
            pltpu.VMEM((PER_W,), jnp.int32),
            pltpu.VMEM((PER_W,), jnp.int32),
            pltpu.VMEM((NBUF, CH, D0), jnp.float32),
            pltpu.VMEM((NBUF, CH, D1), jnp.float32),
            pltpu.VMEM((NBUF, CH, D2), jnp.float32),
            pltpu.SemaphoreType.DMA,
            pltpu.SemaphoreType.DMA,
        ],
    )(_make_sc_body(half))


_sc_calls = [_make_sc_call(0), _make_sc_call(1)]

BB = 128
NBLK = BH // BB


def _tc_body0(g_ref, mt_ref, o_ref):
    for bi in range(BB):
        m = mt_ref[:, bi]
        o_ref[bi] = g_ref[bi, pl.ds(0, L), :] * m[:, None]


def _tc_body1(g_ref, mt_ref, prev_ref, o_ref):
    for bi in range(BB):
        m = mt_ref[:, bi]
        o_ref[bi] = g_ref[bi, pl.ds(0, L), :] * m[:, None]


_tc_call0 = pl.pallas_call(
    _tc_body0,
    grid=(NBLK,),
    in_specs=[
        pl.BlockSpec((BB, LP, DOUT), lambda i: (i, 0, 0)),
        pl.BlockSpec((L, BB), lambda i: (0, i)),
    ],
    out_specs=pl.BlockSpec((BB, L, DOUT), lambda i: (i, 0, 0)),
    out_shape=jax.ShapeDtypeStruct((B, L, DOUT), jnp.float32),
)

_tc_call1 = pl.pallas_call(
    _tc_body1,
    grid=(NBLK,),
    in_specs=[
        pl.BlockSpec((BB, LP, DOUT), lambda i: (i, 0, 0)),
        pl.BlockSpec((L, BB), lambda i: (0, i + NBLK)),
        pl.BlockSpec(memory_space=pl.ANY),
    ],
    out_specs=pl.BlockSpec((BB, L, DOUT), lambda i: (i + NBLK, 0, 0)),
    out_shape=jax.ShapeDtypeStruct((B, L, DOUT), jnp.float32),
    input_output_aliases={2: 0},
)


def kernel(positional_ids_0, positional_ids_1, positional_ids_2,
           attention_mask, W0, W1, W2):
    ids0 = positional_ids_0.reshape(N).astype(jnp.int32)
    ids1 = positional_ids_1.reshape(N).astype(jnp.int32)
    ids2 = positional_ids_2.reshape(N).astype(jnp.int32)
    mt = attention_mask.T
    g0 = _sc_calls[0](ids0, ids1, ids2, W0, W1, W2)
    g1 = _sc_calls[1](ids0, ids1, ids2, W0, W1, W2)
    out = _tc_call0(g0, mt)
    return _tc_call1(g1, mt, out)


# speedup vs baseline: 1.0910x; 1.0176x over previous
import functools

import jax
import jax.numpy as jnp
from jax import lax
from jax.experimental import pallas as pl
from jax.experimental.pallas import tpu as pltpu
from jax.experimental.pallas import tpu_sc as plsc

B, L = 4096, 50
LP = 56
D0, D1, D2 = 64, 32, 32
DOUT = D0 + D1 + D2
N = B * L

_info = plsc.get_sparse_core_info()
NC, NS, LANES = _info.num_cores, _info.num_subcores, _info.num_lanes
NW = NC * NS
BATCH_PER_W = B // NW
PER_W = N // NW
CB = 8
CH = CB * L
N_CHUNKS = BATCH_PER_W // CB
NBUF = 2


def _sc_body(ids0_hbm, ids1_hbm, ids2_hbm, w0_hbm, w1_hbm, w2_hbm,
             out_hbm, idx0_v, idx1_v, idx2_v, e0_v, e1_v, e2_v, gsem, wsem):
    wid = lax.axis_index("s") * NC + lax.axis_index("c")
    batch_w = wid * BATCH_PER_W
    tok_w = batch_w * L

    pltpu.sync_copy(ids0_hbm.at[pl.ds(tok_w, PER_W)], idx0_v)
    pltpu.sync_copy(ids1_hbm.at[pl.ds(tok_w, PER_W)], idx1_v)
    pltpu.sync_copy(ids2_hbm.at[pl.ds(tok_w, PER_W)], idx2_v)

    def fire_gathers(ci, s):
        rows = pl.ds(ci * CH, CH)
        pltpu.async_copy(w0_hbm.at[idx0_v.at[rows]], e0_v.at[s], gsem)
        pltpu.async_copy(w1_hbm.at[idx1_v.at[rows]], e1_v.at[s], gsem)
        pltpu.async_copy(w2_hbm.at[idx2_v.at[rows]], e2_v.at[s], gsem)

    def wait_gathers(s):
        pltpu.make_async_copy(w0_hbm.at[pl.ds(0, CH)], e0_v.at[s], gsem).wait()
        pltpu.make_async_copy(w1_hbm.at[pl.ds(0, CH)], e1_v.at[s], gsem).wait()
        pltpu.make_async_copy(w1_hbm.at[pl.ds(0, CH)], e2_v.at[s], gsem).wait()

    def batch_writes(ci, s):
        b0 = batch_w + ci * CB
        for bi in range(CB):
            rows = pl.ds(bi * L, L)
            yield (e0_v.at[s, rows], out_hbm.at[b0 + bi, pl.ds(0, L),
                                                pl.ds(0, D0)])
            yield (e1_v.at[s, rows], out_hbm.at[b0 + bi, pl.ds(0, L),
                                                pl.ds(D0, D1)])
            yield (e2_v.at[s, rows], out_hbm.at[b0 + bi, pl.ds(0, L),
                                                pl.ds(D0 + D1, D2)])

    def fire_writes(ci, s):
        for src, dst in batch_writes(ci, s):
            pltpu.async_copy(src, dst, wsem)

    def drain_writes(ci, s):
        for src, dst in batch_writes(ci, s):
            pltpu.make_async_copy(src, dst, wsem).wait()

    fire_gathers(0, 0)

    def pair_body(p, carry):
        for s in range(NBUF):
            ci = p * NBUF + s
            wait_gathers(s)
            fire_writes(ci, s)

            @pl.when(ci > 0)
            def _():
                drain_writes(ci - 1, s ^ 1)

            @pl.when(ci + 1 < N_CHUNKS)
            def _():
                fire_gathers(ci + 1, s ^ 1)
        return carry

    lax.fori_loop(0, N_CHUNKS // NBUF, pair_body, 0)
    drain_writes(N_CHUNKS - 1, (N_CHUNKS - 1) % NBUF)


_sc_call = functools.partial(
    pl.kernel,
    out_type=jax.ShapeDtypeStruct((B, LP, DOUT), jnp.float32),
    mesh=plsc.VectorSubcoreMesh(core_axis_name="c", subcore_axis_name="s"),
    compiler_params=pltpu.CompilerParams(use_tc_tiling_on_sc=False),
    scratch_types=[
        pltpu.VMEM((PER_W,), jnp.int32),
        pltpu.VMEM((PER_W,), jnp.int32),
        pltpu.VMEM((PER_W,), jnp.int32),
        pltpu.VMEM((NBUF, CH, D0), jnp.float32),
        pltpu.VMEM((NBUF, CH, D1), jnp.float32),
        pltpu.VMEM((NBUF, CH, D2), jnp.float32),
        pltpu.SemaphoreType.DMA,
        pltpu.SemaphoreType.DMA,
    ],
)(_sc_body)

BB = 64


def _tc_body(g_ref, mt_ref, o_ref):
    for sub in range(2):
        @pl.when(pl.program_id(0) % 2 == sub)
        def _():
            for bi in range(BB):
                m = mt_ref[:, sub * BB + bi]
                o_ref[bi] = g_ref[bi, pl.ds(0, L), :] * m[:, None]


_tc_call = pl.pallas_call(
    _tc_body,
    grid=(B // BB,),
    in_specs=[
        pl.BlockSpec((BB, LP, DOUT), lambda i: (i, 0, 0)),
        pl.BlockSpec((L, 128), lambda i: (0, i // 2)),
    ],
    out_specs=pl.BlockSpec((BB, L, DOUT), lambda i: (i, 0, 0)),
    out_shape=jax.ShapeDtypeStruct((B, L, DOUT), jnp.float32),
)


def kernel(positional_ids_0, positional_ids_1, positional_ids_2,
           attention_mask, W0, W1, W2):
    ids0 = positional_ids_0.reshape(N).astype(jnp.int32)
    ids1 = positional_ids_1.reshape(N).astype(jnp.int32)
    ids2 = positional_ids_2.reshape(N).astype(jnp.int32)
    gathered = _sc_call(ids0, ids1, ids2, W0, W1, W2)
    return _tc_call(gathered, attention_mask.T)


# speedup vs baseline: 1.1399x; 1.0448x over previous
import functools

import jax
import jax.numpy as jnp
from jax import lax
from jax.experimental import pallas as pl
from jax.experimental.pallas import tpu as pltpu
from jax.experimental.pallas import tpu_sc as plsc

B, L = 4096, 50
LP = 56
D0, D1, D2 = 64, 32, 32
DOUT = D0 + D1 + D2
N = B * L

_info = plsc.get_sparse_core_info()
NC, NS, LANES = _info.num_cores, _info.num_subcores, _info.num_lanes
NW = NC * NS
BATCH_PER_W = B // NW
PER_W = N // NW
CB = 8
CH = CB * L
N_CHUNKS = BATCH_PER_W // CB
NBUF = 2


def _sc_body(ids0_hbm, ids1_hbm, ids2_hbm, w0_hbm, w1_hbm, w2_hbm,
             out_hbm, idx0_v, idx1_v, idx2_v, e0_v, e1_v, e2_v, gsem, wsem):
    wid = lax.axis_index("s") * NC + lax.axis_index("c")
    batch_w = wid * BATCH_PER_W
    tok_w = batch_w * L

    pltpu.sync_copy(ids0_hbm.at[pl.ds(tok_w, PER_W)], idx0_v)
    pltpu.sync_copy(ids1_hbm.at[pl.ds(tok_w, PER_W)], idx1_v)
    pltpu.sync_copy(ids2_hbm.at[pl.ds(tok_w, PER_W)], idx2_v)

    def fire_gathers(ci, s):
        rows = pl.ds(ci * CH, CH)
        pltpu.async_copy(w0_hbm.at[idx0_v.at[rows]], e0_v.at[s], gsem)
        pltpu.async_copy(w1_hbm.at[idx1_v.at[rows]], e1_v.at[s], gsem)
        pltpu.async_copy(w2_hbm.at[idx2_v.at[rows]], e2_v.at[s], gsem)

    def wait_gathers(s):
        pltpu.make_async_copy(w0_hbm.at[pl.ds(0, CH)], e0_v.at[s], gsem).wait()
        pltpu.make_async_copy(w1_hbm.at[pl.ds(0, CH)], e1_v.at[s], gsem).wait()
        pltpu.make_async_copy(w1_hbm.at[pl.ds(0, CH)], e2_v.at[s], gsem).wait()

    def batch_writes(ci, s):
        b0 = batch_w + ci * CB
        for bi in range(CB):
            rows = pl.ds(bi * L, L)
            yield (e0_v.at[s, rows], out_hbm.at[b0 + bi, pl.ds(0, L),
                                                pl.ds(0, D0)])
            yield (e1_v.at[s, rows], out_hbm.at[b0 + bi, pl.ds(0, L),
                                                pl.ds(D0, D1)])
            yield (e2_v.at[s, rows], out_hbm.at[b0 + bi, pl.ds(0, L),
                                                pl.ds(D0 + D1, D2)])

    def fire_writes(ci, s):
        for src, dst in batch_writes(ci, s):
            pltpu.async_copy(src, dst, wsem)

    def drain_writes(ci, s):
        for src, dst in batch_writes(ci, s):
            pltpu.make_async_copy(src, dst, wsem).wait()

    fire_gathers(0, 0)

    def pair_body(p, carry):
        for s in range(NBUF):
            ci = p * NBUF + s
            wait_gathers(s)
            fire_writes(ci, s)

            @pl.when(ci > 0)
            def _():
                drain_writes(ci - 1, s ^ 1)

            @pl.when(ci + 1 < N_CHUNKS)
            def _():
                fire_gathers(ci + 1, s ^ 1)
        return carry

    lax.fori_loop(0, N_CHUNKS // NBUF, pair_body, 0)
    drain_writes(N_CHUNKS - 1, (N_CHUNKS - 1) % NBUF)


_sc_call = functools.partial(
    pl.kernel,
    out_type=jax.ShapeDtypeStruct((B, LP, DOUT), jnp.float32),
    mesh=plsc.VectorSubcoreMesh(core_axis_name="c", subcore_axis_name="s"),
    compiler_params=pltpu.CompilerParams(use_tc_tiling_on_sc=False),
    scratch_types=[
        pltpu.VMEM((PER_W,), jnp.int32),
        pltpu.VMEM((PER_W,), jnp.int32),
        pltpu.VMEM((PER_W,), jnp.int32),
        pltpu.VMEM((NBUF, CH, D0), jnp.float32),
        pltpu.VMEM((NBUF, CH, D1), jnp.float32),
        pltpu.VMEM((NBUF, CH, D2), jnp.float32),
        pltpu.SemaphoreType.DMA,
        pltpu.SemaphoreType.DMA,
    ],
)(_sc_body)

BB = 128


def _tc_body(g_ref, mt_ref, o_ref):
    for bi in range(BB):
        m = mt_ref[:, bi]
        o_ref[bi] = g_ref[bi, pl.ds(0, L), :] * m[:, None]


_tc_call = pl.pallas_call(
    _tc_body,
    grid=(B // BB,),
    in_specs=[
        pl.BlockSpec((BB, LP, DOUT), lambda i: (i, 0, 0)),
        pl.BlockSpec((L, BB), lambda i: (0, i)),
    ],
    out_specs=pl.BlockSpec((BB, L, DOUT), lambda i: (i, 0, 0)),
    out_shape=jax.ShapeDtypeStruct((B, L, DOUT), jnp.float32),
)


def kernel(positional_ids_0, positional_ids_1, positional_ids_2,
           attention_mask, W0, W1, W2):
    ids0 = positional_ids_0.reshape(N).astype(jnp.int32)
    ids1 = positional_ids_1.reshape(N).astype(jnp.int32)
    ids2 = positional_ids_2.reshape(N).astype(jnp.int32)
    gathered = _sc_call(ids0, ids1, ids2, W0, W1, W2)
    return _tc_call(gathered, attention_mask.T)
